# baseline (speedup 1.0000x reference)
import functools

import jax
import jax.numpy as jnp
from jax import lax
from jax.experimental import pallas as pl
from jax.experimental.pallas import tpu as pltpu
from jax.experimental.pallas import tpu_sc as plsc

BATCH = 16384
DIM = 32

NC = 2
NS = 16
NW = NC * NS
B_PER_W = BATCH // NW


@functools.cache
def _build():
    mesh = plsc.VectorSubcoreMesh(core_axis_name="c", subcore_axis_name="s")

    @functools.partial(
        pl.kernel,
        mesh=mesh,
        out_type=jax.ShapeDtypeStruct((BATCH, DIM), jnp.float32),
        scratch_types=[
            pltpu.VMEM((B_PER_W,), jnp.int32),
            pltpu.VMEM((B_PER_W, DIM), jnp.float32),
            pltpu.SemaphoreType.DMA,
        ],
        compiler_params=pltpu.CompilerParams(use_tc_tiling_on_sc=False),
    )
    def gather_kernel(idx_hbm, table_hbm, out_hbm, idx_v, rows_v, sem):
        wid = lax.axis_index("s") * NC + lax.axis_index("c")
        base = wid * B_PER_W
        pltpu.sync_copy(idx_hbm.at[pl.ds(base, B_PER_W)], idx_v)
        pltpu.async_copy(table_hbm.at[idx_v], rows_v, sem).wait()
        pltpu.sync_copy(rows_v, out_hbm.at[pl.ds(base, B_PER_W)])

    return gather_kernel


def kernel(domains, embedding_table):
    return _build()(domains.astype(jnp.int32), embedding_table)


# speedup vs baseline: 3.8863x; 3.8863x over previous
import functools

import jax
import jax.numpy as jnp
from jax import lax
from jax.experimental import pallas as pl
from jax.experimental.pallas import tpu as pltpu
from jax.experimental.pallas import tpu_sc as plsc

BATCH = 16384
DIM = 32
NCOLS = 1000000

NC = 2
NS = 16
NW = NC * NS
B_PER_W = BATCH // NW
NBUF = 8


@functools.cache
def _build():
    mesh = plsc.VectorSubcoreMesh(core_axis_name="c", subcore_axis_name="s")

    @functools.partial(
        pl.kernel,
        mesh=mesh,
        out_type=jax.ShapeDtypeStruct((BATCH * DIM,), jnp.float32),
        scratch_types=[
            pltpu.VMEM((B_PER_W,), jnp.int32),
            pltpu.VMEM((NBUF, DIM, 128), jnp.float32),
            pltpu.VMEM((B_PER_W * DIM,), jnp.float32),
            pltpu.SemaphoreType.DMA((NBUF,)),
        ],
        compiler_params=pltpu.CompilerParams(
            use_tc_tiling_on_sc=True, needs_layout_passes=False
        ),
    )
    def gather_kernel(idx_hbm, xt_hbm, out_hbm, idx_v, blk_v, rows_v, sems):
        wid = lax.axis_index("s") * NC + lax.axis_index("c")
        base = wid * B_PER_W
        pltpu.sync_copy(idx_hbm.at[pl.ds(base, B_PER_W)], idx_v)

        lane = lax.iota(jnp.int32, 16)
        c_lo = lane
        c_hi = lane + 16

        def read_idx(j):
            v = idx_v[pl.ds(pl.multiple_of((j >> 4) * 16, 16), 16)]
            return jnp.sum(jnp.where(lane == (j & 15), v, 0))

        def fire(j, slot):
            i = read_idx(j)
            off = pl.multiple_of((i >> 7) * 128, 128)
            pltpu.async_copy(
                xt_hbm.at[:, pl.ds(off, 128)], blk_v.at[slot], sems.at[slot]
            )

        def drain(j, slot):
            i = read_idx(j)
            m = jnp.full((16,), i & 127, jnp.int32)
            pltpu.make_async_copy(
                xt_hbm.at[:, pl.ds(0, 128)], blk_v.at[slot], sems.at[slot]
            ).wait()
            v_lo = plsc.load_gather(blk_v.at[slot], [c_lo, m])
            v_hi = plsc.load_gather(blk_v.at[slot], [c_hi, m])
            o = pl.multiple_of(j * DIM, 16)
            rows_v[pl.ds(o, 16)] = v_lo
            rows_v[pl.ds(o + 16, 16)] = v_hi

        for b in range(NBUF):
            fire(b, b)

        def body(k, carry):
            j0 = k * NBUF
            for b in range(NBUF):
                drain(j0 + b, b)
                fire(j0 + b + NBUF, b)
            return carry

        lax.fori_loop(0, (B_PER_W - NBUF) // NBUF, body, 0)

        for b in range(NBUF):
            drain(B_PER_W - NBUF + b, b)

        pltpu.sync_copy(rows_v, out_hbm.at[pl.ds(base * DIM, B_PER_W * DIM)])

    return gather_kernel


def kernel(domains, embedding_table):
    flat = _build()(domains.astype(jnp.int32), embedding_table.T)
    return flat.reshape(BATCH, DIM)


# speedup vs baseline: 4.3863x; 1.1287x over previous
import functools

import jax
import jax.numpy as jnp
from jax import lax
from jax.experimental import pallas as pl
from jax.experimental.pallas import tpu as pltpu
from jax.experimental.pallas import tpu_sc as plsc

BATCH = 16384
DIM = 32
NCOLS = 1000000
NTILES = (NCOLS + 127) // 128

NC = 2
NS = 16
NW = NC * NS
OWN = (NTILES + NW - 1) // NW
CHUNK = 8
NCHUNK = (OWN + CHUNK - 1) // CHUNK
IDX_STAGE = 1024


@functools.cache
def _build():
    mesh = plsc.VectorSubcoreMesh(core_axis_name="c", subcore_axis_name="s")

    @functools.partial(
        pl.kernel,
        mesh=mesh,
        out_type=jax.ShapeDtypeStruct((BATCH * DIM,), jnp.float32),
        scratch_types=[
            pltpu.VMEM((IDX_STAGE,), jnp.int32),
            pltpu.VMEM((BATCH + 16,), jnp.int32),
            pltpu.VMEM((BATCH + 16,), jnp.int32),
            pltpu.VMEM((2 * CHUNK, DIM, 128), jnp.float32),
            pltpu.VMEM((8 * DIM,), jnp.float32),
            pltpu.SemaphoreType.DMA((2 * CHUNK,)),
            pltpu.SemaphoreType.DMA((8,)),
        ],
        compiler_params=pltpu.CompilerParams(
            use_tc_tiling_on_sc=True, needs_layout_passes=False
        ),
    )
    def gather_kernel(
        idx_hbm, xt_hbm, out_hbm, idx_v, list_v, clist_v, blk_v, tmp_v, sems, wsems
    ):
        wid = lax.axis_index("s") * NC + lax.axis_index("c")
        lo_tile = wid * OWN
        lane = lax.iota(jnp.int32, 16)
        c_hi = lane + 16

        def stage(st, cnt):
            pltpu.sync_copy(idx_hbm.at[pl.ds(st * IDX_STAGE, IDX_STAGE)], idx_v)

            def scan(vi, cnt):
                v = idx_v[pl.ds(pl.multiple_of(vi * 16, 16), 16)]
                loc = (v >> 7) - lo_tile
                mask = (loc >= 0) & (loc < OWN)
                b = st * IDX_STAGE + vi * 16 + lane
                key = (loc << 21) | ((v & 127) << 14) | b
                plsc.store_compressed(list_v.at[pl.ds(cnt, 16)], key, mask=mask)
                return cnt + jnp.sum(jnp.where(mask, 1, 0))

            return lax.fori_loop(0, IDX_STAGE // 16, scan, cnt)

        cnt = lax.fori_loop(0, BATCH // IDX_STAGE, stage, 0)

        def fire(cn, b):
            t = jnp.minimum(lo_tile + cn * CHUNK + b, NTILES - 1)
            off = pl.multiple_of(t * 128, 128)
            slot = (cn & 1) * CHUNK + b
            pltpu.async_copy(
                xt_hbm.at[:, pl.ds(off, 128)], blk_v.at[slot], sems.at[slot]
            )

        for b in range(CHUNK):
            fire(0, b)

        def chunk_body(c, wcnt):
            for b in range(CHUNK):
                slot = (c & 1) * CHUNK + b
                pltpu.make_async_copy(
                    xt_hbm.at[:, pl.ds(0, 128)], blk_v.at[slot], sems.at[slot]
                ).wait()

            @pl.when(c + 1 < NCHUNK)
            def _():
                for b in range(CHUNK):
                    fire(c + 1, b)

            t0 = c * CHUNK

            def sel(vi, ccnt):
                kv = list_v[pl.ds(pl.multiple_of(vi * 16, 16), 16)]
                tloc = kv >> 21
                mask = (
                    (tloc >= t0)
                    & (tloc < t0 + CHUNK)
                    & (vi * 16 + lane < cnt)
                )
                plsc.store_compressed(clist_v.at[pl.ds(ccnt, 16)], kv, mask=mask)
                return ccnt + jnp.sum(jnp.where(mask, 1, 0))

            ccnt = lax.fori_loop(0, (cnt + 15) >> 4, sel, 0)

            def proc(j, wcnt):
                kv = clist_v[pl.ds(pl.multiple_of((j >> 4) * 16, 16), 16)]
                key = jnp.sum(jnp.where(lane == (j & 15), kv, 0))
                slot = (c & 1) * CHUNK + ((key >> 21) & (CHUNK - 1))
                m = jnp.full((16,), (key >> 14) & 127, jnp.int32)
                b_out = key & 16383
                ws = wcnt & 7

                @pl.when(wcnt >= 8)
                def _():
                    pltpu.make_async_copy(
                        tmp_v.at[pl.ds(0, DIM)],
                        out_hbm.at[pl.ds(0, DIM)],
                        wsems.at[ws],
                    ).wait()

                v_lo = plsc.load_gather(blk_v.at[slot], [lane, m])
                v_hi = plsc.load_gather(blk_v.at[slot], [c_hi, m])
                o = pl.multiple_of(ws * DIM, 16)
                tmp_v[pl.ds(o, 16)] = v_lo
                tmp_v[pl.ds(o + 16, 16)] = v_hi
                pltpu.async_copy(
                    tmp_v.at[pl.ds(o, DIM)],
                    out_hbm.at[pl.ds(b_out * DIM, DIM)],
                    wsems.at[ws],
                )
                return wcnt + 1

            return lax.fori_loop(0, ccnt, proc, wcnt)

        wcnt = lax.fori_loop(0, NCHUNK, chunk_body, 0)

        for k in range(8):

            @pl.when(k < wcnt)
            def _():
                pltpu.make_async_copy(
                    tmp_v.at[pl.ds(0, DIM)],
                    out_hbm.at[pl.ds(0, DIM)],
                    wsems.at[k],
                ).wait()

    return gather_kernel


def kernel(domains, embedding_table):
    flat = _build()(domains.astype(jnp.int32), embedding_table.T)
    return flat.reshape(BATCH, DIM)


# speedup vs baseline: 5.2155x; 1.1890x over previous
import functools

import jax
import jax.numpy as jnp
from jax import lax
from jax.experimental import pallas as pl
from jax.experimental.pallas import tpu as pltpu
from jax.experimental.pallas import tpu_sc as plsc

BATCH = 16384
DIM = 32
NCOLS = 1000000
NTILES = (NCOLS + 127) // 128

NC = 2
NS = 16
NW = NC * NS
OWN = (NTILES + NW - 1) // NW
CHUNK = 8
NCHUNK = (OWN + CHUNK - 1) // CHUNK
IDX_STAGE = 1024


@functools.cache
def _build():
    mesh = plsc.VectorSubcoreMesh(core_axis_name="c", subcore_axis_name="s")

    @functools.partial(
        pl.kernel,
        mesh=mesh,
        out_type=jax.ShapeDtypeStruct((BATCH * DIM,), jnp.float32),
        scratch_types=[
            pltpu.VMEM((2, IDX_STAGE), jnp.int32),
            pltpu.VMEM((BATCH + 16,), jnp.int32),
            pltpu.VMEM((BATCH + 16,), jnp.int32),
            pltpu.VMEM((2 * CHUNK, DIM, 128), jnp.float32),
            pltpu.VMEM((8 * DIM,), jnp.float32),
            pltpu.SemaphoreType.DMA((2 * CHUNK,)),
            pltpu.SemaphoreType.DMA((8,)),
            pltpu.SemaphoreType.DMA((2,)),
        ],
        compiler_params=pltpu.CompilerParams(
            use_tc_tiling_on_sc=True, needs_layout_passes=False
        ),
    )
    def gather_kernel(
        idx_hbm,
        xt_hbm,
        out_hbm,
        idx_v,
        list_v,
        clist_v,
        blk_v,
        tmp_v,
        sems,
        wsems,
        isems,
    ):
        wid = lax.axis_index("s") * NC + lax.axis_index("c")
        lo_tile = wid * OWN
        lane = lax.iota(jnp.int32, 16)
        c_hi = lane + 16

        def fire_idx(st):
            pltpu.async_copy(
                idx_hbm.at[pl.ds(st * IDX_STAGE, IDX_STAGE)],
                idx_v.at[st & 1],
                isems.at[st & 1],
            )

        fire_idx(0)

        def stage(st, cnt):
            pltpu.make_async_copy(
                idx_hbm.at[pl.ds(0, IDX_STAGE)], idx_v.at[st & 1], isems.at[st & 1]
            ).wait()

            @pl.when(st + 1 < BATCH // IDX_STAGE)
            def _():
                fire_idx(st + 1)

            def scan(vi, cnt):
                v = idx_v[st & 1, pl.ds(pl.multiple_of(vi * 16, 16), 16)]
                loc = (v >> 7) - lo_tile
                mask = (loc >= 0) & (loc < OWN)
                b = st * IDX_STAGE + vi * 16 + lane
                key = (loc << 21) | ((v & 127) << 14) | b
                plsc.store_compressed(list_v.at[pl.ds(cnt, 16)], key, mask=mask)
                return cnt + jnp.sum(jnp.where(mask, 1, 0))

            return lax.fori_loop(0, IDX_STAGE // 16, scan, cnt)

        cnt = lax.fori_loop(0, BATCH // IDX_STAGE, stage, 0)

        def fire(cn, b):
            t = jnp.minimum(lo_tile + cn * CHUNK + b, NTILES - 1)
            off = pl.multiple_of(t * 128, 128)
            slot = (cn & 1) * CHUNK + b
            pltpu.async_copy(
                xt_hbm.at[:, pl.ds(off, 128)], blk_v.at[slot], sems.at[slot]
            )

        for b in range(CHUNK):
            fire(0, b)
        for b in range(CHUNK):
            fire(1, b)

        def chunk_body(c, wcnt):
            for b in range(CHUNK):
                slot = (c & 1) * CHUNK + b
                pltpu.make_async_copy(
                    xt_hbm.at[:, pl.ds(0, 128)], blk_v.at[slot], sems.at[slot]
                ).wait()

            t0 = c * CHUNK

            def sel(vi, ccnt):
                kv = list_v[pl.ds(pl.multiple_of(vi * 16, 16), 16)]
                tloc = kv >> 21
                mask = (
                    (tloc >= t0)
                    & (tloc < t0 + CHUNK)
                    & (vi * 16 + lane < cnt)
                )
                plsc.store_compressed(clist_v.at[pl.ds(ccnt, 16)], kv, mask=mask)
                return ccnt + jnp.sum(jnp.where(mask, 1, 0))

            ccnt = lax.fori_loop(0, (cnt + 15) >> 4, sel, 0)

            def proc(j, wcnt):
                kv = clist_v[pl.ds(pl.multiple_of((j >> 4) * 16, 16), 16)]
                key = jnp.sum(jnp.where(lane == (j & 15), kv, 0))
                slot = (c & 1) * CHUNK + ((key >> 21) & (CHUNK - 1))
                m = jnp.full((16,), (key >> 14) & 127, jnp.int32)
                b_out = key & 16383
                ws = wcnt & 7

                @pl.when(wcnt >= 8)
                def _():
                    pltpu.make_async_copy(
                        tmp_v.at[pl.ds(0, DIM)],
                        out_hbm.at[pl.ds(0, DIM)],
                        wsems.at[ws],
                    ).wait()

                v_lo = plsc.load_gather(blk_v.at[slot], [lane, m])
                v_hi = plsc.load_gather(blk_v.at[slot], [c_hi, m])
                o = pl.multiple_of(ws * DIM, 16)
                tmp_v[pl.ds(o, 16)] = v_lo
                tmp_v[pl.ds(o + 16, 16)] = v_hi
                pltpu.async_copy(
                    tmp_v.at[pl.ds(o, DIM)],
                    out_hbm.at[pl.ds(b_out * DIM, DIM)],
                    wsems.at[ws],
                )
                return wcnt + 1

            wcnt = lax.fori_loop(0, ccnt, proc, wcnt)

            @pl.when(c + 2 < NCHUNK)
            def _():
                for b in range(CHUNK):
                    fire(c + 2, b)

            return wcnt

        wcnt = lax.fori_loop(0, NCHUNK, chunk_body, 0)

        for k in range(8):

            @pl.when(k < wcnt)
            def _():
                pltpu.make_async_copy(
                    tmp_v.at[pl.ds(0, DIM)],
                    out_hbm.at[pl.ds(0, DIM)],
                    wsems.at[k],
                ).wait()

    return gather_kernel


def kernel(domains, embedding_table):
    flat = _build()(domains.astype(jnp.int32), embedding_table.T)
    return flat.reshape(BATCH, DIM)


# speedup vs baseline: 5.2380x; 1.0043x over previous
import functools

import jax
import jax.numpy as jnp
from jax import lax
from jax.experimental import pallas as pl
from jax.experimental.pallas import tpu as pltpu
from jax.experimental.pallas import tpu_sc as plsc

BATCH = 16384
DIM = 32
NCOLS = 1000000
NTILES = (NCOLS + 127) // 128

NC = 2
NS = 16
NW = NC * NS
OWN = (NTILES + NW - 1) // NW
CHUNK = 8
NCHUNK = (OWN + CHUNK - 1) // CHUNK
IDX_STAGE = 1024


@functools.cache
def _build():
    mesh = plsc.VectorSubcoreMesh(core_axis_name="c", subcore_axis_name="s")

    @functools.partial(
        pl.kernel,
        mesh=mesh,
        out_type=jax.ShapeDtypeStruct((BATCH * DIM,), jnp.float32),
        scratch_types=[
            pltpu.VMEM((2, IDX_STAGE), jnp.int32),
            pltpu.VMEM((BATCH + 16,), jnp.int32),
            pltpu.VMEM((BATCH + 16,), jnp.int32),
            pltpu.VMEM((256,), jnp.int32),
            pltpu.VMEM((256,), jnp.int32),
            pltpu.VMEM((256 + 16,), jnp.int32),
            pltpu.VMEM((2 * CHUNK, DIM, 128), jnp.float32),
            pltpu.VMEM((8 * DIM,), jnp.float32),
            pltpu.SemaphoreType.DMA((2 * CHUNK,)),
            pltpu.SemaphoreType.DMA((8,)),
            pltpu.SemaphoreType.DMA((2,)),
        ],
        compiler_params=pltpu.CompilerParams(
            use_tc_tiling_on_sc=True, needs_layout_passes=False
        ),
    )
    def gather_kernel(
        idx_hbm,
        xt_hbm,
        out_hbm,
        idx_v,
        list_v,
        clist_v,
        need_v,
        pos_v,
        tiles_v,
        blk_v,
        tmp_v,
        sems,
        wsems,
        isems,
    ):
        wid = lax.axis_index("s") * NC + lax.axis_index("c")
        lo_tile = wid * OWN
        lane = lax.iota(jnp.int32, 16)
        c_hi = lane + 16
        zeros16 = jnp.zeros((16,), jnp.int32)
        ones16 = jnp.ones((16,), jnp.int32)
        for w in range(16):
            need_v[pl.ds(16 * w, 16)] = zeros16

        def fire_idx(st):
            pltpu.async_copy(
                idx_hbm.at[pl.ds(st * IDX_STAGE, IDX_STAGE)],
                idx_v.at[st & 1],
                isems.at[st & 1],
            )

        fire_idx(0)

        def stage(st, cnt):
            pltpu.make_async_copy(
                idx_hbm.at[pl.ds(0, IDX_STAGE)], idx_v.at[st & 1], isems.at[st & 1]
            ).wait()

            @pl.when(st + 1 < BATCH // IDX_STAGE)
            def _():
                fire_idx(st + 1)

            def scan(vi, cnt):
                v = idx_v[st & 1, pl.ds(pl.multiple_of(vi * 16, 16), 16)]
                loc = (v >> 7) - lo_tile
                mask = (loc >= 0) & (loc < OWN)
                b = st * IDX_STAGE + vi * 16 + lane
                key = (loc << 21) | ((v & 127) << 14) | b
                plsc.store_compressed(list_v.at[pl.ds(cnt, 16)], key, mask=mask)
                plsc.store_scatter(need_v, [loc], ones16, mask=mask)
                return cnt + jnp.sum(jnp.where(mask, 1, 0))

            return lax.fori_loop(0, IDX_STAGE // 16, scan, cnt)

        cnt = lax.fori_loop(0, BATCH // IDX_STAGE, stage, 0)

        def build_tiles(w, ntile):
            lv = need_v[pl.ds(16 * w, 16)]
            pos_v[pl.ds(16 * w, 16)] = ntile + plsc.cumsum(lv) - lv
            plsc.store_compressed(
                tiles_v.at[pl.ds(ntile, 16)], 16 * w + lane, mask=lv > 0
            )
            return ntile + jnp.sum(lv)

        ntile = lax.fori_loop(0, 16, build_tiles, 0)
        nchunk = jnp.maximum((ntile + CHUNK - 1) >> 3, 2)

        def fire(cn, b):
            k = cn * CHUNK + b
            tv = tiles_v[pl.ds(pl.multiple_of((k >> 4) * 16, 16), 16)]
            tl = jnp.sum(jnp.where(lane == (k & 15), tv, 0))
            t = jnp.minimum(
                lo_tile + jnp.clip(tl, 0, 255), NTILES - 1
            )
            off = pl.multiple_of(t * 128, 128)
            slot = (cn & 1) * CHUNK + b
            pltpu.async_copy(
                xt_hbm.at[:, pl.ds(off, 128)], blk_v.at[slot], sems.at[slot]
            )

        for b in range(CHUNK):
            fire(0, b)
        for b in range(CHUNK):
            fire(1, b)

        def chunk_body(c, wcnt):
            for b in range(CHUNK):
                slot = (c & 1) * CHUNK + b
                pltpu.make_async_copy(
                    xt_hbm.at[:, pl.ds(0, 128)], blk_v.at[slot], sems.at[slot]
                ).wait()

            p0 = c * CHUNK

            def sel(vi, ccnt):
                kv = list_v[pl.ds(pl.multiple_of(vi * 16, 16), 16)]
                pos = plsc.load_gather(pos_v, [(kv >> 21) & 255])
                mask = (
                    (pos >= p0)
                    & (pos < p0 + CHUNK)
                    & (vi * 16 + lane < cnt)
                )
                kv2 = (pos << 21) | (kv & ((1 << 21) - 1))
                plsc.store_compressed(clist_v.at[pl.ds(ccnt, 16)], kv2, mask=mask)
                return ccnt + jnp.sum(jnp.where(mask, 1, 0))

            ccnt = lax.fori_loop(0, (cnt + 15) >> 4, sel, 0)

            def proc(j, wcnt):
                kv = clist_v[pl.ds(pl.multiple_of((j >> 4) * 16, 16), 16)]
                key = jnp.sum(jnp.where(lane == (j & 15), kv, 0))
                slot = (c & 1) * CHUNK + ((key >> 21) & (CHUNK - 1))
                m = jnp.full((16,), (key >> 14) & 127, jnp.int32)
                b_out = key & 16383
                ws = wcnt & 7

                @pl.when(wcnt >= 8)
                def _():
                    pltpu.make_async_copy(
                        tmp_v.at[pl.ds(0, DIM)],
                        out_hbm.at[pl.ds(0, DIM)],
                        wsems.at[ws],
                    ).wait()

                v_lo = plsc.load_gather(blk_v.at[slot], [lane, m])
                v_hi = plsc.load_gather(blk_v.at[slot], [c_hi, m])
                o = pl.multiple_of(ws * DIM, 16)
                tmp_v[pl.ds(o, 16)] = v_lo
                tmp_v[pl.ds(o + 16, 16)] = v_hi
                pltpu.async_copy(
                    tmp_v.at[pl.ds(o, DIM)],
                    out_hbm.at[pl.ds(b_out * DIM, DIM)],
                    wsems.at[ws],
                )
                return wcnt + 1

            wcnt = lax.fori_loop(0, ccnt, proc, wcnt)

            @pl.when(c + 2 < nchunk)
            def _():
                for b in range(CHUNK):
                    fire(c + 2, b)

            return wcnt

        wcnt = lax.fori_loop(0, nchunk, chunk_body, 0)

        for k in range(8):

            @pl.when(k < wcnt)
            def _():
                pltpu.make_async_copy(
                    tmp_v.at[pl.ds(0, DIM)],
                    out_hbm.at[pl.ds(0, DIM)],
                    wsems.at[k],
                ).wait()

    return gather_kernel


def kernel(domains, embedding_table):
    flat = _build()(domains.astype(jnp.int32), embedding_table.T)
    return flat.reshape(BATCH, DIM)
